# Initial kernel scaffold; baseline (speedup 1.0000x reference)
import jax
import jax.numpy as jnp
from jax.experimental import pallas as pl


def kernel(x):
    raise NotImplementedError("write your pallas kernel here")


# speedup vs baseline: 3.3272x; 3.3272x over previous
import functools

import jax
import jax.numpy as jnp
from jax import lax
from jax.experimental import pallas as pl
from jax.experimental.pallas import tpu as pltpu
from jax.experimental.pallas import tpu_sc as plsc

_NCATS = 100000
_BATCH = 1024
_HIST = 200
_LANES = 16

_info = plsc.get_sparse_core_info()
_NC = _info.num_cores
_NW = _NC * _info.num_subcores
_ROWS_PER_W = _BATCH // _NW

_WINDOWS = [j * _LANES for j in range(_HIST // _LANES)]
if _HIST % _LANES:
    _WINDOWS.append(_HIST - _LANES)

_mesh = plsc.VectorSubcoreMesh(core_axis_name="c", subcore_axis_name="s")


@functools.partial(
    pl.kernel,
    mesh=_mesh,
    out_type=jax.ShapeDtypeStruct((_BATCH, _NCATS), jnp.float32),
    scratch_types=[
        pltpu.VMEM((_NCATS,), jnp.float32),
        pltpu.VMEM((_HIST,), jnp.int32),
    ],
    compiler_params=pltpu.CompilerParams(needs_layout_passes=False),
)
def _encode(x_hbm, out_hbm, row_v, idx_v):
    wid = lax.axis_index("s") * _NC + lax.axis_index("c")
    zeros16 = jnp.zeros((_LANES,), jnp.float32)
    ones16 = jnp.ones((_LANES,), jnp.float32)

    def zero_body(i, carry):
        base = i * (10 * _LANES)
        for j in range(10):
            row_v[pl.ds(base + j * _LANES, _LANES)] = zeros16
        return carry

    lax.fori_loop(0, _NCATS // (10 * _LANES), zero_body, 0)

    def row_body(r, carry):
        row = wid * _ROWS_PER_W + r
        pltpu.sync_copy(x_hbm.at[row], idx_v)
        for off in _WINDOWS:
            plsc.store_scatter(row_v, [idx_v[pl.ds(off, _LANES)]], ones16)
        pltpu.sync_copy(row_v, out_hbm.at[row])
        for off in _WINDOWS:
            plsc.store_scatter(row_v, [idx_v[pl.ds(off, _LANES)]], zeros16)
        return carry

    lax.fori_loop(0, _ROWS_PER_W, row_body, 0)


def kernel(x):
    return _encode(x)
